# baseline (speedup 1.0000x reference)
import functools

import jax
import jax.numpy as jnp
from jax import lax
from jax.experimental import pallas as pl
from jax.experimental.pallas import tpu as pltpu
from jax.experimental.pallas import tpu_sc as plsc

_N = 10000
_E = 320000
_H = 128
_D = 32
_G = 64

_NC = 2
_NS = 16
_NW = _NC * _NS
_EPT = _E // _NW
_CE = 40
_NCH = _EPT // _CE
_NPAD = 10240
_NPT = _NPAD // _NS

_BN = 1000
_BE = 2000

_LP = (6.944574454170738e-05, 0.9962619482337949, -0.466442438627571,
       0.21866548366222835, -0.055459313742084085)


def _log1p_poly(t):
  acc = jnp.full_like(t, _LP[4])
  for c in (_LP[3], _LP[2], _LP[1], _LP[0]):
    acc = acc * t + c
  return acc


def _pack2(a, b):
  ah = lax.bitcast_convert_type(a.astype(jnp.bfloat16), jnp.uint16)
  bh = lax.bitcast_convert_type(b.astype(jnp.bfloat16), jnp.uint16)
  w = ah.astype(jnp.uint32) | (bh.astype(jnp.uint32) << 16)
  return lax.bitcast_convert_type(w, jnp.float32)


def _unpack2(v):
  return plsc.unpack(plsc.bitcast(v, jnp.bfloat16),
                     format=plsc.PackFormat.INTERLEAVED)


def _cprep_body(ea_ref, we_ref, be_ref, *c_refs):
  c = jnp.dot(ea_ref[...], we_ref[...],
              preferred_element_type=jnp.float32) + be_ref[...]
  for l in range(3):
    c_refs[l][...] = _pack2(c[:, (2 * l) * _H:(2 * l + 1) * _H],
                            c[:, (2 * l + 1) * _H:(2 * l + 2) * _H])


_cprep = pl.pallas_call(
    _cprep_body,
    grid=(_E // _BE,),
    in_specs=[
        pl.BlockSpec((_BE, _D), lambda i: (i, 0)),
        pl.BlockSpec((_D, 6 * _H), lambda i: (0, 0)),
        pl.BlockSpec((1, 6 * _H), lambda i: (0, 0)),
    ],
    out_specs=[pl.BlockSpec((_BE, _H), lambda i: (i, 0))] * 3,
    out_shape=[jax.ShapeDtypeStruct((_E, _H), jnp.float32)] * 3,
)


def _project_and_pack(h, w_ref, pd_ref, ps_ref):
  r = jnp.dot(h, w_ref[...], preferred_element_type=jnp.float32)
  pd_ref[...] = _pack2(r[:, 0:_H], r[:, _H:2 * _H])
  ps_ref[...] = _pack2(r[:, 2 * _H:3 * _H], r[:, 3 * _H:4 * _H])


def _proj1_body(x_ref, emb_ref, w_ref, h_ref, pd_ref, ps_ref):
  iot = lax.broadcasted_iota(jnp.int32, (_BN, _H), 1)
  oh = (x_ref[...] == iot).astype(jnp.float32)
  h = jnp.dot(oh, emb_ref[...], preferred_element_type=jnp.float32)
  h_ref[...] = h
  _project_and_pack(h, w_ref, pd_ref, ps_ref)


_proj1 = pl.pallas_call(
    _proj1_body,
    grid=(_N // _BN,),
    in_specs=[
        pl.BlockSpec((_BN, 1), lambda i: (i, 0)),
        pl.BlockSpec((_H, _H), lambda i: (0, 0)),
        pl.BlockSpec((_H, 4 * _H), lambda i: (0, 0)),
    ],
    out_specs=[pl.BlockSpec((_BN, _H), lambda i: (i, 0))] * 3,
    out_shape=[jax.ShapeDtypeStruct((_N, _H), jnp.float32)] * 3,
)


def _proj_body(h_ref, p_ref, w_ref, hn_ref, pd_ref, ps_ref):
  hn = jnp.maximum(h_ref[...] + p_ref[0] + p_ref[1], 0.0)
  hn_ref[...] = hn
  _project_and_pack(hn, w_ref, pd_ref, ps_ref)


_proj = pl.pallas_call(
    _proj_body,
    grid=(_N // _BN,),
    in_specs=[
        pl.BlockSpec((_BN, _H), lambda i: (i, 0)),
        pl.BlockSpec((_NC, _BN, _H), lambda i: (0, i, 0)),
        pl.BlockSpec((_H, 4 * _H), lambda i: (0, 0)),
    ],
    out_specs=[pl.BlockSpec((_BN, _H), lambda i: (i, 0))] * 3,
    out_shape=[jax.ShapeDtypeStruct((_N, _H), jnp.float32)] * 3,
)


def _pool_body(h_ref, p_ref, b_ref, wl_ref, bl_ref, o_ref, acc, cnt):
  i = pl.program_id(0)

  @pl.when(i == 0)
  def _():
    acc[...] = jnp.zeros_like(acc)
    cnt[...] = jnp.zeros_like(cnt)

  h3 = jnp.maximum(h_ref[...] + p_ref[0] + p_ref[1], 0.0)
  iot = lax.broadcasted_iota(jnp.int32, (_G, _BN), 0)
  pt = (b_ref[0] == iot).astype(jnp.float32)
  acc[...] += jnp.dot(pt, h3, preferred_element_type=jnp.float32)
  cnt[...] += jnp.dot(pt, jnp.ones_like(h3), preferred_element_type=jnp.float32)

  @pl.when(i == pl.num_programs(0) - 1)
  def _():
    pooled = acc[...] / jnp.maximum(cnt[...], 1.0)
    o_ref[...] = jnp.dot(pooled, wl_ref[...],
                         preferred_element_type=jnp.float32) + bl_ref[...]


_pool = pl.pallas_call(
    _pool_body,
    grid=(_N // _BN,),
    in_specs=[
        pl.BlockSpec((_BN, _H), lambda i: (i, 0)),
        pl.BlockSpec((_NC, _BN, _H), lambda i: (0, i, 0)),
        pl.BlockSpec((1, 1, _BN), lambda i: (i, 0, 0)),
        pl.BlockSpec((_H, _H), lambda i: (0, 0)),
        pl.BlockSpec((1, _H), lambda i: (0, 0)),
    ],
    out_specs=pl.BlockSpec((_G, _H), lambda i: (0, 0)),
    out_shape=jax.ShapeDtypeStruct((_G, _H), jnp.float32),
    scratch_shapes=[
        pltpu.VMEM((_G, _H), jnp.float32),
        pltpu.VMEM((_G, _H), jnp.float32),
    ],
)


def _edge_body(pd_hbm, ps_hbm, cc_hbm, dst_hbm, src_hbm, zero_hbm, out_hbm,
               agg, dstb, srcb, pdb, psb, ccb, mb, semf, semg, sems):
  cid = lax.axis_index("c")
  sid = lax.axis_index("s")
  wid = sid * _NC + cid
  base = wid * _EPT

  pltpu.sync_copy(zero_hbm, agg.at[pl.ds(sid * _NPT, _NPT)])
  plsc.subcore_barrier()

  def front(i, s, b):
    off = base + i * _CE
    pltpu.async_copy(dst_hbm.at[pl.ds(off, _CE)], dstb.at[s], semf.at[b])
    pltpu.async_copy(src_hbm.at[pl.ds(off, _CE)], srcb.at[s], semf.at[b])
    pltpu.async_copy(cc_hbm.at[pl.ds(off, _CE)], ccb.at[b], semf.at[b])

  def wait_front(i, s, b):
    off = base + i * _CE
    pltpu.make_async_copy(dst_hbm.at[pl.ds(off, _CE)], dstb.at[s],
                          semf.at[b]).wait()
    pltpu.make_async_copy(src_hbm.at[pl.ds(off, _CE)], srcb.at[s],
                          semf.at[b]).wait()
    pltpu.make_async_copy(cc_hbm.at[pl.ds(off, _CE)], ccb.at[b],
                          semf.at[b]).wait()

  def gathers(s, b):
    pltpu.async_copy(pd_hbm.at[dstb.at[s]], pdb.at[b], semg.at[b])
    pltpu.async_copy(ps_hbm.at[srcb.at[s]], psb.at[b], semg.at[b])

  def wait_gathers(s, b):
    pltpu.make_async_copy(pd_hbm.at[dstb.at[s]], pdb.at[b],
                          semg.at[b]).wait()
    pltpu.make_async_copy(ps_hbm.at[srcb.at[s]], psb.at[b],
                          semg.at[b]).wait()

  def compute_scatter(c, s, b):
    @pl.when(c >= 2)
    def _():
      pltpu.make_async_copy(mb.at[b], agg.at[dstb.at[s]], sems.at[b]).wait()

    @plsc.parallel_loop(0, _CE, unroll=2)
    def edge(e):
      for k in range(_H // 16):
        sl = pl.ds(k * 16, 16)
        vsum = (plsc.bitcast(pdb[b, e, sl], jnp.bfloat16)
                + plsc.bitcast(psb[b, e, sl], jnp.bfloat16)
                + plsc.bitcast(ccb[b, e, sl], jnp.bfloat16))
        f, s2 = plsc.unpack(vsum, format=plsc.PackFormat.INTERLEAVED)
        sig = 1.0 / (1.0 + jnp.exp(-f))
        t = jnp.exp(-jnp.abs(s2))
        sp = jnp.maximum(s2, 0.0) + _log1p_poly(t)
        mb[b, e, sl] = sig * sp
    pltpu.async_copy(mb.at[b], agg.at[dstb.at[s]], sems.at[b], add=True)

  front(0, 0, 0)
  wait_front(0, 0, 0)
  gathers(0, 0)
  front(1, 1, 1)

  def pair(p, carry):
    c0 = 2 * p
    s0 = 2 * lax.rem(p, 2)
    s1 = s0 + 1
    sn0 = 2 - s0
    sn1 = sn0 + 1

    wait_gathers(s0, 0)

    @pl.when(c0 + 1 < _NCH)
    def _():
      wait_front(c0 + 1, s1, 1)
      gathers(s1, 1)

    compute_scatter(c0, s0, 0)

    @pl.when(c0 + 2 < _NCH)
    def _():
      front(c0 + 2, sn0, 0)

    @pl.when(c0 + 1 < _NCH)
    def _():
      wait_gathers(s1, 1)

      @pl.when(c0 + 2 < _NCH)
      def _():
        wait_front(c0 + 2, sn0, 0)
        gathers(sn0, 0)

      compute_scatter(c0 + 1, s1, 1)

      @pl.when(c0 + 3 < _NCH)
      def _():
        front(c0 + 3, sn1, 1)

    return carry

  lax.fori_loop(0, (_NCH + 1) // 2, pair, 0)
  pltpu.make_async_copy(mb.at[0], agg.at[dstb.at[0]], sems.at[0]).wait()
  pltpu.make_async_copy(mb.at[1], agg.at[dstb.at[1]], sems.at[1]).wait()
  plsc.subcore_barrier()
  pltpu.sync_copy(agg.at[pl.ds(sid * _NPT, _NPT)],
                  out_hbm.at[cid, pl.ds(sid * _NPT, _NPT)])


@functools.cache
def _edge_pass_fn():
  return functools.partial(
      pl.kernel,
      out_type=jax.ShapeDtypeStruct((_NC, _NPAD, _H), jnp.float32),
      mesh=plsc.VectorSubcoreMesh(core_axis_name="c", subcore_axis_name="s",
                                  num_cores=_NC, num_subcores=_NS),
      compiler_params=pltpu.CompilerParams(needs_layout_passes=False),
      scratch_types=[
          pltpu.VMEM_SHARED((_NPAD, _H), jnp.float32),
          pltpu.VMEM((4, _CE), jnp.int32),
          pltpu.VMEM((4, _CE), jnp.int32),
          pltpu.VMEM((2, _CE, _H), jnp.float32),
          pltpu.VMEM((2, _CE, _H), jnp.float32),
          pltpu.VMEM((2, _CE, _H), jnp.float32),
          pltpu.VMEM((2, _CE, _H), jnp.float32),
          pltpu.SemaphoreType.DMA((2,)),
          pltpu.SemaphoreType.DMA((2,)),
          pltpu.SemaphoreType.DMA((2,)),
      ],
  )(_edge_body)


def _edge_pass(*args):
  return _edge_pass_fn()(*args)


def kernel(x, edge_index, edge_attr, batch, emb,
           Wf1, bf1, Ws1, bs1, Wf2, bf2, Ws2, bs2, Wf3, bf3, Ws3, bs3,
           Wlin, blin):
  f32 = jnp.float32
  x2 = x.astype(jnp.int32).reshape(_N, 1)
  src = edge_index[0].astype(jnp.int32)
  dst = edge_index[1].astype(jnp.int32)
  b2 = batch.astype(jnp.int32).reshape(_N // _BN, 1, _BN)
  emb_pad = jnp.zeros((_H, _H), f32).at[:emb.shape[0]].set(emb)
  zeros = jnp.zeros((_NPT, _H), f32)

  def wall(Wf, Ws):
    return jnp.concatenate(
        [Wf[:_H], Ws[:_H], Wf[_H:2 * _H], Ws[_H:2 * _H]], axis=1)

  w1, w2, w3 = wall(Wf1, Ws1), wall(Wf2, Ws2), wall(Wf3, Ws3)
  we_all = jnp.concatenate(
      [Wf1[2 * _H:], Ws1[2 * _H:], Wf2[2 * _H:], Ws2[2 * _H:],
       Wf3[2 * _H:], Ws3[2 * _H:]], axis=1)
  be_all = jnp.concatenate([bf1, bs1, bf2, bs2, bf3, bs3]).reshape(1, 6 * _H)

  c1, c2, c3 = _cprep(edge_attr, we_all, be_all)

  h0, pd, ps = _proj1(x2, emb_pad, w1)
  p = _edge_pass(pd, ps, c1, dst, src, zeros)
  h1, pd, ps = _proj(h0, p, w2)
  p = _edge_pass(pd, ps, c2, dst, src, zeros)
  h2, pd, ps = _proj(h1, p, w3)
  p = _edge_pass(pd, ps, c3, dst, src, zeros)
  return _pool(h2, p, b2, Wlin, blin.reshape(1, _H))


# speedup vs baseline: 1.3076x; 1.3076x over previous
import functools

import jax
import jax.numpy as jnp
from jax import lax
from jax.experimental import pallas as pl
from jax.experimental.pallas import tpu as pltpu
from jax.experimental.pallas import tpu_sc as plsc

_N = 10000
_E = 320000
_H = 128
_D = 32
_G = 64

_NC = 2
_NS = 16
_NW = _NC * _NS
_EPT = _E // _NW
_CE = 40
_NCH = _EPT // _CE
_NPAD = 10240
_NPT = _NPAD // _NS

_BN = 1000
_BE = 2000

_LP = (6.944574454170738e-05, 0.9962619482337949, -0.466442438627571,
       0.21866548366222835, -0.055459313742084085)


def _log1p_poly(t):
  acc = jnp.full_like(t, _LP[4])
  for c in (_LP[3], _LP[2], _LP[1], _LP[0]):
    acc = acc * t + c
  return acc


def _pack2(a, b):
  ah = lax.bitcast_convert_type(a.astype(jnp.bfloat16), jnp.uint16)
  bh = lax.bitcast_convert_type(b.astype(jnp.bfloat16), jnp.uint16)
  w = ah.astype(jnp.uint32) | (bh.astype(jnp.uint32) << 16)
  return lax.bitcast_convert_type(w, jnp.float32)


def _unpack2(v):
  return plsc.unpack(plsc.bitcast(v, jnp.bfloat16),
                     format=plsc.PackFormat.INTERLEAVED)


def _cprep_body(ea_ref, we_ref, be_ref, c_ref):
  c = jnp.dot(ea_ref[...], we_ref[...],
              preferred_element_type=jnp.float32) + be_ref[...]
  c_ref[...] = _pack2(c[:, 0:_H], c[:, _H:2 * _H])


_cprep = pl.pallas_call(
    _cprep_body,
    grid=(_E // _BE,),
    in_specs=[
        pl.BlockSpec((_BE, _D), lambda i: (i, 0)),
        pl.BlockSpec((_D, 2 * _H), lambda i: (0, 0)),
        pl.BlockSpec((1, 2 * _H), lambda i: (0, 0)),
    ],
    out_specs=pl.BlockSpec((_BE, _H), lambda i: (i, 0)),
    out_shape=jax.ShapeDtypeStruct((_E, _H), jnp.float32),
)


def _project_and_pack(h, w_ref, pd_ref, ps_ref):
  r = jnp.dot(h, w_ref[...], preferred_element_type=jnp.float32)
  pd_ref[...] = _pack2(r[:, 0:_H], r[:, _H:2 * _H])
  ps_ref[...] = _pack2(r[:, 2 * _H:3 * _H], r[:, 3 * _H:4 * _H])


def _proj1_body(x_ref, emb_ref, w_ref, h_ref, pd_ref, ps_ref):
  iot = lax.broadcasted_iota(jnp.int32, (_BN, _H), 1)
  oh = (x_ref[...] == iot).astype(jnp.float32)
  h = jnp.dot(oh, emb_ref[...], preferred_element_type=jnp.float32)
  h_ref[...] = h
  _project_and_pack(h, w_ref, pd_ref, ps_ref)


_proj1 = pl.pallas_call(
    _proj1_body,
    grid=(_N // _BN,),
    in_specs=[
        pl.BlockSpec((_BN, 1), lambda i: (i, 0)),
        pl.BlockSpec((_H, _H), lambda i: (0, 0)),
        pl.BlockSpec((_H, 4 * _H), lambda i: (0, 0)),
    ],
    out_specs=[pl.BlockSpec((_BN, _H), lambda i: (i, 0))] * 3,
    out_shape=[jax.ShapeDtypeStruct((_N, _H), jnp.float32)] * 3,
)


def _proj_body(h_ref, p_ref, w_ref, hn_ref, pd_ref, ps_ref):
  hn = jnp.maximum(h_ref[...] + p_ref[0] + p_ref[1], 0.0)
  hn_ref[...] = hn
  _project_and_pack(hn, w_ref, pd_ref, ps_ref)


_proj = pl.pallas_call(
    _proj_body,
    grid=(_N // _BN,),
    in_specs=[
        pl.BlockSpec((_BN, _H), lambda i: (i, 0)),
        pl.BlockSpec((_NC, _BN, _H), lambda i: (0, i, 0)),
        pl.BlockSpec((_H, 4 * _H), lambda i: (0, 0)),
    ],
    out_specs=[pl.BlockSpec((_BN, _H), lambda i: (i, 0))] * 3,
    out_shape=[jax.ShapeDtypeStruct((_N, _H), jnp.float32)] * 3,
)


def _pool_body(h_ref, p_ref, b_ref, wl_ref, bl_ref, o_ref, acc, cnt):
  i = pl.program_id(0)

  @pl.when(i == 0)
  def _():
    acc[...] = jnp.zeros_like(acc)
    cnt[...] = jnp.zeros_like(cnt)

  h3 = jnp.maximum(h_ref[...] + p_ref[0] + p_ref[1], 0.0)
  iot = lax.broadcasted_iota(jnp.int32, (_G, _BN), 0)
  pt = (b_ref[0] == iot).astype(jnp.float32)
  acc[...] += jnp.dot(pt, h3, preferred_element_type=jnp.float32)
  cnt[...] += jnp.dot(pt, jnp.ones_like(h3), preferred_element_type=jnp.float32)

  @pl.when(i == pl.num_programs(0) - 1)
  def _():
    pooled = acc[...] / jnp.maximum(cnt[...], 1.0)
    o_ref[...] = jnp.dot(pooled, wl_ref[...],
                         preferred_element_type=jnp.float32) + bl_ref[...]


_pool = pl.pallas_call(
    _pool_body,
    grid=(_N // _BN,),
    in_specs=[
        pl.BlockSpec((_BN, _H), lambda i: (i, 0)),
        pl.BlockSpec((_NC, _BN, _H), lambda i: (0, i, 0)),
        pl.BlockSpec((1, 1, _BN), lambda i: (i, 0, 0)),
        pl.BlockSpec((_H, _H), lambda i: (0, 0)),
        pl.BlockSpec((1, _H), lambda i: (0, 0)),
    ],
    out_specs=pl.BlockSpec((_G, _H), lambda i: (0, 0)),
    out_shape=jax.ShapeDtypeStruct((_G, _H), jnp.float32),
    scratch_shapes=[
        pltpu.VMEM((_G, _H), jnp.float32),
        pltpu.VMEM((_G, _H), jnp.float32),
    ],
)


def _edge_body(pd_hbm, ps_hbm, cc_hbm, dst_hbm, src_hbm, zero_hbm, out_hbm,
               agg, dstb, srcb, pdb, psb, ccb, mb, semf, semg, sems):
  cid = lax.axis_index("c")
  sid = lax.axis_index("s")
  wid = sid * _NC + cid
  base = wid * _EPT

  pltpu.sync_copy(zero_hbm, agg.at[pl.ds(sid * _NPT, _NPT)])
  plsc.subcore_barrier()

  def front(i, s, b):
    off = base + i * _CE
    pltpu.async_copy(dst_hbm.at[pl.ds(off, _CE)], dstb.at[s], semf.at[b])
    pltpu.async_copy(src_hbm.at[pl.ds(off, _CE)], srcb.at[s], semf.at[b])
    pltpu.async_copy(cc_hbm.at[pl.ds(off, _CE)], ccb.at[b], semf.at[b])

  def wait_front(i, s, b):
    off = base + i * _CE
    pltpu.make_async_copy(dst_hbm.at[pl.ds(off, _CE)], dstb.at[s],
                          semf.at[b]).wait()
    pltpu.make_async_copy(src_hbm.at[pl.ds(off, _CE)], srcb.at[s],
                          semf.at[b]).wait()
    pltpu.make_async_copy(cc_hbm.at[pl.ds(off, _CE)], ccb.at[b],
                          semf.at[b]).wait()

  def gathers(s, b):
    pltpu.async_copy(pd_hbm.at[dstb.at[s]], pdb.at[b], semg.at[b])
    pltpu.async_copy(ps_hbm.at[srcb.at[s]], psb.at[b], semg.at[b])

  def wait_gathers(s, b):
    pltpu.make_async_copy(pd_hbm.at[dstb.at[s]], pdb.at[b],
                          semg.at[b]).wait()
    pltpu.make_async_copy(ps_hbm.at[srcb.at[s]], psb.at[b],
                          semg.at[b]).wait()

  def compute_scatter(c, s, b):
    @pl.when(c >= 2)
    def _():
      pltpu.make_async_copy(mb.at[b], agg.at[dstb.at[s]], sems.at[b]).wait()

    def edge(e, c2):
      for k in range(_H // 16):
        sl = pl.ds(k * 16, 16)
        vsum = (plsc.bitcast(pdb[b, e, sl], jnp.bfloat16)
                + plsc.bitcast(psb[b, e, sl], jnp.bfloat16)
                + plsc.bitcast(ccb[b, e, sl], jnp.bfloat16))
        f, s2 = plsc.unpack(vsum, format=plsc.PackFormat.INTERLEAVED)
        sig = 1.0 / (1.0 + jnp.exp(-f))
        t = jnp.exp(-jnp.abs(s2))
        sp = jnp.maximum(s2, 0.0) + _log1p_poly(t)
        mb[b, e, sl] = sig * sp
      return c2

    lax.fori_loop(0, _CE, edge, 0)
    pltpu.async_copy(mb.at[b], agg.at[dstb.at[s]], sems.at[b], add=True)

  front(0, 0, 0)
  wait_front(0, 0, 0)
  gathers(0, 0)
  front(1, 1, 1)

  def pair(p, carry):
    c0 = 2 * p
    s0 = 2 * lax.rem(p, 2)
    s1 = s0 + 1
    sn0 = 2 - s0
    sn1 = sn0 + 1

    wait_gathers(s0, 0)

    @pl.when(c0 + 1 < _NCH)
    def _():
      wait_front(c0 + 1, s1, 1)
      gathers(s1, 1)

    compute_scatter(c0, s0, 0)

    @pl.when(c0 + 2 < _NCH)
    def _():
      front(c0 + 2, sn0, 0)

    @pl.when(c0 + 1 < _NCH)
    def _():
      wait_gathers(s1, 1)

      @pl.when(c0 + 2 < _NCH)
      def _():
        wait_front(c0 + 2, sn0, 0)
        gathers(sn0, 0)

      compute_scatter(c0 + 1, s1, 1)

      @pl.when(c0 + 3 < _NCH)
      def _():
        front(c0 + 3, sn1, 1)

    return carry

  lax.fori_loop(0, (_NCH + 1) // 2, pair, 0)
  pltpu.make_async_copy(mb.at[0], agg.at[dstb.at[0]], sems.at[0]).wait()
  pltpu.make_async_copy(mb.at[1], agg.at[dstb.at[1]], sems.at[1]).wait()
  plsc.subcore_barrier()
  pltpu.sync_copy(agg.at[pl.ds(sid * _NPT, _NPT)],
                  out_hbm.at[cid, pl.ds(sid * _NPT, _NPT)])


@functools.cache
def _edge_pass_fn():
  return functools.partial(
      pl.kernel,
      out_type=jax.ShapeDtypeStruct((_NC, _NPAD, _H), jnp.float32),
      mesh=plsc.VectorSubcoreMesh(core_axis_name="c", subcore_axis_name="s",
                                  num_cores=_NC, num_subcores=_NS),
      compiler_params=pltpu.CompilerParams(needs_layout_passes=False),
      scratch_types=[
          pltpu.VMEM_SHARED((_NPAD, _H), jnp.float32),
          pltpu.VMEM((4, _CE), jnp.int32),
          pltpu.VMEM((4, _CE), jnp.int32),
          pltpu.VMEM((2, _CE, _H), jnp.float32),
          pltpu.VMEM((2, _CE, _H), jnp.float32),
          pltpu.VMEM((2, _CE, _H), jnp.float32),
          pltpu.VMEM((2, _CE, _H), jnp.float32),
          pltpu.SemaphoreType.DMA((2,)),
          pltpu.SemaphoreType.DMA((2,)),
          pltpu.SemaphoreType.DMA((2,)),
      ],
  )(_edge_body)


def _edge_pass(*args):
  return _edge_pass_fn()(*args)


def kernel(x, edge_index, edge_attr, batch, emb,
           Wf1, bf1, Ws1, bs1, Wf2, bf2, Ws2, bs2, Wf3, bf3, Ws3, bs3,
           Wlin, blin):
  f32 = jnp.float32
  x2 = x.astype(jnp.int32).reshape(_N, 1)
  src = edge_index[0].astype(jnp.int32)
  dst = edge_index[1].astype(jnp.int32)
  b2 = batch.astype(jnp.int32).reshape(_N // _BN, 1, _BN)
  emb_pad = jnp.zeros((_H, _H), f32).at[:emb.shape[0]].set(emb)
  zeros = jnp.zeros((_NPT, _H), f32)

  def wall(Wf, Ws):
    return jnp.concatenate(
        [Wf[:_H], Ws[:_H], Wf[_H:2 * _H], Ws[_H:2 * _H]], axis=1)

  w1, w2, w3 = wall(Wf1, Ws1), wall(Wf2, Ws2), wall(Wf3, Ws3)

  def cprep(Wf, Ws, bf, bs):
    we = jnp.concatenate([Wf[2 * _H:], Ws[2 * _H:]], axis=1)
    be = jnp.concatenate([bf, bs]).reshape(1, 2 * _H)
    return _cprep(edge_attr, we, be)

  c1 = cprep(Wf1, Ws1, bf1, bs1)
  c2 = cprep(Wf2, Ws2, bf2, bs2)
  c3 = cprep(Wf3, Ws3, bf3, bs3)

  h0, pd, ps = _proj1(x2, emb_pad, w1)
  p = _edge_pass(pd, ps, c1, dst, src, zeros)
  h1, pd, ps = _proj(h0, p, w2)
  p = _edge_pass(pd, ps, c2, dst, src, zeros)
  h2, pd, ps = _proj(h1, p, w3)
  p = _edge_pass(pd, ps, c3, dst, src, zeros)
  return _pool(h2, p, b2, Wlin, blin.reshape(1, _H))
